# Initial kernel scaffold; baseline (speedup 1.0000x reference)
import jax
import jax.numpy as jnp
from jax.experimental import pallas as pl


def kernel(user_embeddings, item_embeddings, rows, cols):
    raise NotImplementedError("write your pallas kernel here")


# speedup vs baseline: 1.9263x; 1.9263x over previous
import functools

import jax
import jax.numpy as jnp
from jax import lax
from jax.experimental import pallas as pl
from jax.experimental.pallas import tpu as pltpu
from jax.experimental.pallas import tpu_sc as plsc

NU = 5000
NI = 5000
D = 128
P = 5120
NP = 2 * P
NL = 4
E = 160000
EPT = 10240
EPC = 128
RPT = 320
RB = 64
NSUB = 16

_mesh = plsc.VectorSubcoreMesh(core_axis_name="c", subcore_axis_name="s")


def _hsum(x):
    idx = lax.iota(jnp.int32, 16)
    for sh in (8, 4, 2, 1):
        perm = lax.bitwise_and(idx + sh, 15)
        x = x + x.at[perm].get(mode="promise_in_bounds")
    return x


def _layer_body(emb_in, ego, dsti, srci, valf, acc_in, emb_out, acc_out,
                acc_s, dbuf, sbuf, vbuf, rbuf, abuf, ebuf, cbuf, sem):
    c = lax.axis_index("c")
    s = lax.axis_index("s")

    def zrow(i, _):
        z = jnp.zeros((16,), jnp.float32)
        for r in range(8):
            abuf[i, pl.ds(16 * r, 16)] = z
        return 0
    lax.fori_loop(0, RB, zrow, 0)

    def zcp(b, _):
        pltpu.sync_copy(abuf, acc_s.at[pl.ds(s * RPT + b * RB, RB)])
        return 0
    lax.fori_loop(0, RPT // RB, zcp, 0)
    plsc.subcore_barrier()

    ebase = (c * NSUB + s) * EPT

    def chunk(k, _):
        base = ebase + k * EPC
        pltpu.sync_copy(dsti.at[pl.ds(base, EPC)], dbuf)
        pltpu.sync_copy(srci.at[pl.ds(base, EPC)], sbuf)
        pltpu.sync_copy(valf.at[pl.ds(base, EPC)], vbuf)
        pltpu.async_copy(emb_in.at[dbuf], rbuf, sem).wait()

        def scale(g, _):
            vv = vbuf[pl.ds(16 * g, 16)]
            for l in range(16):
                j = 16 * g + l
                v = vv[l]
                for r in range(8):
                    rbuf[j, pl.ds(16 * r, 16)] = rbuf[j, pl.ds(16 * r, 16)] * v
            return 0
        lax.fori_loop(0, EPC // 16, scale, 0)
        pltpu.sync_copy(rbuf, acc_s.at[sbuf], add=True)
        return 0
    lax.fori_loop(0, EPT // EPC, chunk, 0)
    plsc.subcore_barrier()

    def blk(b, _):
        loc = s * RPT + b * RB
        g = c * P + loc
        pltpu.sync_copy(acc_s.at[pl.ds(loc, RB)], abuf)
        pltpu.sync_copy(ego.at[pl.ds(g, RB)], ebuf)
        pltpu.sync_copy(acc_in.at[pl.ds(g, RB)], cbuf)

        def row(i, _):
            dot = jnp.zeros((16,), jnp.float32)
            sa = jnp.zeros((16,), jnp.float32)
            se = jnp.zeros((16,), jnp.float32)
            for r in range(8):
                av = abuf[i, pl.ds(16 * r, 16)]
                ev = ebuf[i, pl.ds(16 * r, 16)]
                dot = dot + av * ev
                sa = sa + av * av
                se = se + ev * ev
            dots = _hsum(dot)
            p = jnp.maximum(_hsum(sa) * _hsum(se), jnp.float32(1e-16))
            ip = lax.bitcast_convert_type(p, jnp.int32)
            iy = jnp.full((16,), 0x5F3759DF, jnp.int32) - \
                lax.shift_right_arithmetic(ip, jnp.full((16,), 1, jnp.int32))
            y = lax.bitcast_convert_type(iy, jnp.float32)
            for _ in range(3):
                y = y * (jnp.float32(1.5) - jnp.float32(0.5) * p * y * y)
            w = dots * y
            for r in range(8):
                av = abuf[i, pl.ds(16 * r, 16)]
                ov = av * w
                abuf[i, pl.ds(16 * r, 16)] = ov
                cbuf[i, pl.ds(16 * r, 16)] = cbuf[i, pl.ds(16 * r, 16)] + ov
            return 0
        lax.fori_loop(0, RB, row, 0)
        pltpu.sync_copy(abuf, emb_out.at[pl.ds(g, RB)])
        pltpu.sync_copy(cbuf, acc_out.at[pl.ds(g, RB)])
        return 0
    lax.fori_loop(0, RPT // RB, blk, 0)


_layer = functools.partial(
    pl.kernel,
    out_type=(
        jax.ShapeDtypeStruct((NP, D), jnp.float32),
        jax.ShapeDtypeStruct((NP, D), jnp.float32),
    ),
    mesh=_mesh,
    scratch_types=[
        pltpu.MemorySpace.VMEM_SHARED((P, D), jnp.float32),
        pltpu.VMEM((EPC,), jnp.int32),
        pltpu.VMEM((EPC,), jnp.int32),
        pltpu.VMEM((EPC,), jnp.float32),
        pltpu.VMEM((EPC, D), jnp.float32),
        pltpu.VMEM((RB, D), jnp.float32),
        pltpu.VMEM((RB, D), jnp.float32),
        pltpu.VMEM((RB, D), jnp.float32),
        pltpu.SemaphoreType.DMA,
    ],
)(_layer_body)


def _pad_side(a, fill):
    a = a.reshape(NSUB, E // NSUB)
    return jnp.pad(a, ((0, 0), (0, EPT - E // NSUB)), constant_values=fill)


def kernel(user_embeddings, item_embeddings, rows, cols):
    row_sum = jnp.zeros((NU,), jnp.float32).at[rows].add(1.0) + 1e-07
    col_sum = jnp.zeros((NI,), jnp.float32).at[cols].add(1.0) + 1e-07
    vals = jax.lax.rsqrt(row_sum)[rows] * jax.lax.rsqrt(col_sum)[cols]

    dsti = jnp.concatenate([
        _pad_side(cols + P, 0).ravel(), _pad_side(rows, 0).ravel()])
    srci = jnp.concatenate([
        _pad_side(rows, P - 1).ravel(), _pad_side(cols, P - 1).ravel()])
    valf = jnp.concatenate([_pad_side(vals, 0.0).ravel()] * 2)

    ego = jnp.zeros((NP, D), jnp.float32)
    ego = ego.at[:NU].set(user_embeddings).at[P:P + NI].set(item_embeddings)

    x = ego
    acc = jnp.zeros((NP, D), jnp.float32)
    for _ in range(NL):
        x, acc = _layer(x, ego, dsti, srci, valf, acc)
    return (acc[:NU], acc[P:P + NI])


# speedup vs baseline: 4.2907x; 2.2274x over previous
import functools

import jax
import jax.numpy as jnp
from jax import lax
from jax.experimental import pallas as pl
from jax.experimental.pallas import tpu as pltpu
from jax.experimental.pallas import tpu_sc as plsc

NU = 5000
NI = 5000
D = 128
P = 5120
NP = 2 * P
NL = 4
E = 160000
EPT = 10240
EPC = 128
NCH = EPT // EPC
RPT = 320
RB = 64
NSUB = 16

_mesh = plsc.VectorSubcoreMesh(core_axis_name="c", subcore_axis_name="s")


def _hsum(x):
    idx = lax.iota(jnp.int32, 16)
    for sh in (8, 4, 2, 1):
        perm = lax.bitwise_and(idx + sh, 15)
        x = x + x.at[perm].get(mode="promise_in_bounds")
    return x


def _nrsqrt(p):
    ip = lax.bitcast_convert_type(p, jnp.int32)
    iy = jnp.full((16,), 0x5F3759DF, jnp.int32) - \
        lax.shift_right_arithmetic(ip, jnp.full((16,), 1, jnp.int32))
    y = lax.bitcast_convert_type(iy, jnp.float32)
    for _ in range(3):
        y = y * (jnp.float32(1.5) - jnp.float32(0.5) * p * y * y)
    return y


def _layer_body(xs_in, ego, dsti, srci, dnode, acc_in, xs_out, acc_out,
                acc_s, dall, sall, rbuf0, rbuf1, abuf, ebuf, cbuf,
                dbufd, dsplat, sem0, sem1):
    c = lax.axis_index("c")
    s = lax.axis_index("s")

    def zrow(i, _):
        z = jnp.zeros((16,), jnp.float32)
        for r in range(8):
            abuf[i, pl.ds(16 * r, 16)] = z
        return 0
    lax.fori_loop(0, RB, zrow, 0)

    def zcp(b, _):
        pltpu.sync_copy(abuf, acc_s.at[pl.ds(s * RPT + b * RB, RB)])
        return 0
    lax.fori_loop(0, RPT // RB, zcp, 0)

    tb = (c * NSUB + s) * NCH
    pltpu.sync_copy(dsti.at[pl.ds(tb, NCH)], dall)
    pltpu.sync_copy(srci.at[pl.ds(tb, NCH)], sall)
    plsc.subcore_barrier()

    pltpu.async_copy(xs_in.at[dall.at[0]], rbuf0, sem0)

    def step(k, _):
        pltpu.make_async_copy(xs_in.at[dall.at[2 * k]], rbuf0, sem0).wait()
        pltpu.async_copy(xs_in.at[dall.at[2 * k + 1]], rbuf1, sem1)
        pltpu.sync_copy(rbuf0, acc_s.at[sall.at[2 * k]], add=True)
        pltpu.make_async_copy(
            xs_in.at[dall.at[2 * k + 1]], rbuf1, sem1).wait()

        @pl.when(k < NCH // 2 - 1)
        def _():
            pltpu.async_copy(xs_in.at[dall.at[2 * k + 2]], rbuf0, sem0)
        pltpu.sync_copy(rbuf1, acc_s.at[sall.at[2 * k + 1]], add=True)
        return 0
    lax.fori_loop(0, NCH // 2, step, 0)
    plsc.subcore_barrier()

    def blk(b, _):
        loc = s * RPT + b * RB
        g = c * P + loc
        pltpu.sync_copy(acc_s.at[pl.ds(loc, RB)], abuf)
        pltpu.sync_copy(ego.at[pl.ds(g, RB)], ebuf)
        pltpu.sync_copy(acc_in.at[pl.ds(g, RB)], cbuf)
        pltpu.sync_copy(dnode.at[pl.ds(g, RB)], dbufd)
        for gg in range(RB // 16):
            dv = dbufd[pl.ds(16 * gg, 16)]
            for l in range(16):
                dsplat[16 * gg + l, :] = jnp.broadcast_to(dv[l], (16,))

        def row(i, _):
            di = dsplat[i]
            dot = jnp.zeros((16,), jnp.float32)
            sa = jnp.zeros((16,), jnp.float32)
            se = jnp.zeros((16,), jnp.float32)
            for r in range(8):
                av = abuf[i, pl.ds(16 * r, 16)] * di
                ev = ebuf[i, pl.ds(16 * r, 16)]
                abuf[i, pl.ds(16 * r, 16)] = av
                dot = dot + av * ev
                sa = sa + av * av
                se = se + ev * ev
            p = jnp.maximum(_hsum(sa) * _hsum(se), jnp.float32(1e-16))
            w = _hsum(dot) * _nrsqrt(p)
            for r in range(8):
                ov = abuf[i, pl.ds(16 * r, 16)] * w
                cbuf[i, pl.ds(16 * r, 16)] = cbuf[i, pl.ds(16 * r, 16)] + ov
                abuf[i, pl.ds(16 * r, 16)] = ov * di
            return 0
        lax.fori_loop(0, RB, row, 0)
        pltpu.sync_copy(abuf, xs_out.at[pl.ds(g, RB)])
        pltpu.sync_copy(cbuf, acc_out.at[pl.ds(g, RB)])
        return 0
    lax.fori_loop(0, RPT // RB, blk, 0)


_layer = functools.partial(
    pl.kernel,
    out_type=(
        jax.ShapeDtypeStruct((NP, D), jnp.float32),
        jax.ShapeDtypeStruct((NP, D), jnp.float32),
    ),
    mesh=_mesh,
    scratch_types=[
        pltpu.MemorySpace.VMEM_SHARED((P, D), jnp.float32),
        pltpu.VMEM((NCH, EPC), jnp.int32),
        pltpu.VMEM((NCH, EPC), jnp.int32),
        pltpu.VMEM((EPC, D), jnp.float32),
        pltpu.VMEM((EPC, D), jnp.float32),
        pltpu.VMEM((RB, D), jnp.float32),
        pltpu.VMEM((RB, D), jnp.float32),
        pltpu.VMEM((RB, D), jnp.float32),
        pltpu.VMEM((RB,), jnp.float32),
        pltpu.VMEM((RB, 16), jnp.float32),
        pltpu.SemaphoreType.DMA,
        pltpu.SemaphoreType.DMA,
    ],
)(_layer_body)


def _pad_side(a, fill):
    a = a.reshape(NSUB, E // NSUB)
    a = jnp.pad(a, ((0, 0), (0, EPT - E // NSUB)), constant_values=fill)
    return a.reshape(NSUB * NCH, EPC)


def kernel(user_embeddings, item_embeddings, rows, cols):
    row_sum = jnp.zeros((NU,), jnp.float32).at[rows].add(1.0) + 1e-07
    col_sum = jnp.zeros((NI,), jnp.float32).at[cols].add(1.0) + 1e-07
    dnode = jnp.zeros((NP,), jnp.float32)
    dnode = dnode.at[:NU].set(jax.lax.rsqrt(row_sum))
    dnode = dnode.at[P:P + NI].set(jax.lax.rsqrt(col_sum))

    dsti = jnp.concatenate([_pad_side(cols + P, 0), _pad_side(rows, 0)])
    srci = jnp.concatenate([_pad_side(rows, P - 1), _pad_side(cols, P - 1)])

    ego = jnp.zeros((NP, D), jnp.float32)
    ego = ego.at[:NU].set(user_embeddings).at[P:P + NI].set(item_embeddings)

    xs = dnode[:, None] * ego
    acc = jnp.zeros((NP, D), jnp.float32)
    for _ in range(NL):
        xs, acc = _layer(xs, ego, dsti, srci, dnode, acc)
    return (acc[:NU], acc[P:P + NI])


# speedup vs baseline: 5.2940x; 1.2338x over previous
import functools

import jax
import jax.numpy as jnp
from jax import lax
from jax.experimental import pallas as pl
from jax.experimental.pallas import tpu as pltpu
from jax.experimental.pallas import tpu_sc as plsc

NU = 5000
NI = 5000
D = 128
P = 5120
NP = 2 * P
NL = 4
E = 160000
EPT = 10240
EPC = 80
NCH = EPT // EPC
RPT = 320
RB = 32
NSUB = 16
NBUF = 4

_mesh = plsc.VectorSubcoreMesh(core_axis_name="c", subcore_axis_name="s")


def _hsum(x):
    idx = lax.iota(jnp.int32, 16)
    for sh in (8, 4, 2, 1):
        perm = lax.bitwise_and(idx + sh, 15)
        x = x + x.at[perm].get(mode="promise_in_bounds")
    return x


def _nrsqrt(p):
    ip = lax.bitcast_convert_type(p, jnp.int32)
    iy = jnp.full((16,), 0x5F3759DF, jnp.int32) - \
        lax.shift_right_arithmetic(ip, jnp.full((16,), 1, jnp.int32))
    y = lax.bitcast_convert_type(iy, jnp.float32)
    for _ in range(3):
        y = y * (jnp.float32(1.5) - jnp.float32(0.5) * p * y * y)
    return y


def _layer_body(xs_in, ego, dsti, srci, dnode, acc_in, xs_out, acc_out,
                acc_s, dall, sall, rbuf0, rbuf1, rbuf2, rbuf3,
                gsem0, gsem1, gsem2, gsem3, ssem0, ssem1, ssem2, ssem3):
    c = lax.axis_index("c")
    s = lax.axis_index("s")

    def zrow(i, _):
        z = jnp.zeros((16,), jnp.float32)
        for r in range(8):
            rbuf0[i, pl.ds(16 * r, 16)] = z
        return 0
    lax.fori_loop(0, RB, zrow, 0)

    def zcp(b, _):
        pltpu.sync_copy(rbuf0.at[pl.ds(0, RB)],
                        acc_s.at[pl.ds(s * RPT + b * RB, RB)])
        return 0
    lax.fori_loop(0, RPT // RB, zcp, 0)

    tb = (c * NSUB + s) * NCH
    pltpu.sync_copy(dsti.at[pl.ds(tb, NCH)], dall)
    pltpu.sync_copy(srci.at[pl.ds(tb, NCH)], sall)
    plsc.subcore_barrier()

    rbufs = (rbuf0, rbuf1, rbuf2, rbuf3)
    gsems = (gsem0, gsem1, gsem2, gsem3)
    ssems = (ssem0, ssem1, ssem2, ssem3)
    for b in range(NBUF):
        pltpu.async_copy(xs_in.at[dall.at[b]], rbufs[b], gsems[b])

    def step(q, _):
        for b in range(NBUF):
            k = NBUF * q + b
            pltpu.make_async_copy(
                xs_in.at[dall.at[k]], rbufs[b], gsems[b]).wait()
            pltpu.async_copy(rbufs[b], acc_s.at[sall.at[k]], ssems[b],
                             add=True)
            pltpu.make_async_copy(
                rbufs[b], acc_s.at[sall.at[k]], ssems[b]).wait()

            @pl.when(k + NBUF < NCH)
            def _():
                pltpu.async_copy(
                    xs_in.at[dall.at[k + NBUF]], rbufs[b], gsems[b])
        return 0
    lax.fori_loop(0, NCH // NBUF, step, 0)
    plsc.subcore_barrier()

    def blk(b, _):
        loc = s * RPT + b * RB
        g = c * P + loc
        pltpu.sync_copy(acc_s.at[pl.ds(loc, RB)], rbuf0.at[pl.ds(0, RB)])
        pltpu.sync_copy(ego.at[pl.ds(g, RB)], rbuf1.at[pl.ds(0, RB)])
        pltpu.sync_copy(acc_in.at[pl.ds(g, RB)], rbuf2.at[pl.ds(0, RB)])
        pltpu.sync_copy(dnode.at[pl.ds(g, RB)], rbuf3.at[0, pl.ds(0, RB)])
        for gg in range(RB // 16):
            dv = rbuf3[0, pl.ds(16 * gg, 16)]
            for l in range(16):
                rbuf3[1 + 16 * gg + l, pl.ds(0, 16)] = \
                    jnp.broadcast_to(dv[l], (16,))

        def row(i, _):
            di = rbuf3[1 + i, pl.ds(0, 16)]
            dot = jnp.zeros((16,), jnp.float32)
            sa = jnp.zeros((16,), jnp.float32)
            se = jnp.zeros((16,), jnp.float32)
            for r in range(8):
                av = rbuf0[i, pl.ds(16 * r, 16)] * di
                ev = rbuf1[i, pl.ds(16 * r, 16)]
                rbuf0[i, pl.ds(16 * r, 16)] = av
                dot = dot + av * ev
                sa = sa + av * av
                se = se + ev * ev
            p = jnp.maximum(_hsum(sa) * _hsum(se), jnp.float32(1e-16))
            w = _hsum(dot) * _nrsqrt(p)
            for r in range(8):
                ov = rbuf0[i, pl.ds(16 * r, 16)] * w
                rbuf2[i, pl.ds(16 * r, 16)] = \
                    rbuf2[i, pl.ds(16 * r, 16)] + ov
                rbuf0[i, pl.ds(16 * r, 16)] = ov * di
            return 0
        lax.fori_loop(0, RB, row, 0)
        pltpu.sync_copy(rbuf0.at[pl.ds(0, RB)], xs_out.at[pl.ds(g, RB)])
        pltpu.sync_copy(rbuf2.at[pl.ds(0, RB)], acc_out.at[pl.ds(g, RB)])
        return 0
    lax.fori_loop(0, RPT // RB, blk, 0)


_layer = functools.partial(
    pl.kernel,
    out_type=(
        jax.ShapeDtypeStruct((NP, D), jnp.float32),
        jax.ShapeDtypeStruct((NP, D), jnp.float32),
    ),
    mesh=_mesh,
    scratch_types=[
        pltpu.MemorySpace.VMEM_SHARED((P, D), jnp.float32),
        pltpu.VMEM((NCH, EPC), jnp.int32),
        pltpu.VMEM((NCH, EPC), jnp.int32),
        pltpu.VMEM((EPC, D), jnp.float32),
        pltpu.VMEM((EPC, D), jnp.float32),
        pltpu.VMEM((EPC, D), jnp.float32),
        pltpu.VMEM((EPC, D), jnp.float32),
        pltpu.SemaphoreType.DMA,
        pltpu.SemaphoreType.DMA,
        pltpu.SemaphoreType.DMA,
        pltpu.SemaphoreType.DMA,
        pltpu.SemaphoreType.DMA,
        pltpu.SemaphoreType.DMA,
        pltpu.SemaphoreType.DMA,
        pltpu.SemaphoreType.DMA,
    ],
)(_layer_body)


def _pre_body(ego, srci, dnode_out, xs0_out,
              deg_s, sall, ones, dbufd, dsplat, ebuf):
    c = lax.axis_index("c")
    s = lax.axis_index("s")
    for gg in range(RB // 16):
        dbufd[pl.ds(16 * gg, 16)] = jnp.zeros((16,), jnp.float32)

    def zcp(b, _):
        pltpu.sync_copy(dbufd, deg_s.at[pl.ds(s * RPT + b * RB, RB)])
        return 0
    lax.fori_loop(0, RPT // RB, zcp, 0)
    for g in range(EPC // 16):
        ones[pl.ds(16 * g, 16)] = jnp.full((16,), 1.0, jnp.float32)
    tb = (c * NSUB + s) * NCH
    pltpu.sync_copy(srci.at[pl.ds(tb, NCH)], sall)
    plsc.subcore_barrier()

    def step(k, _):
        pltpu.sync_copy(ones, deg_s.at[sall.at[k]], add=True)
        return 0
    lax.fori_loop(0, NCH, step, 0)
    plsc.subcore_barrier()

    def blk(b, _):
        loc = s * RPT + b * RB
        g = c * P + loc
        pltpu.sync_copy(deg_s.at[pl.ds(loc, RB)], dbufd)
        pltpu.sync_copy(ego.at[pl.ds(g, RB)], ebuf)
        for gg in range(RB // 16):
            dd = _nrsqrt(dbufd[pl.ds(16 * gg, 16)] + jnp.float32(1e-07))
            dbufd[pl.ds(16 * gg, 16)] = dd
            for l in range(16):
                dsplat[16 * gg + l, :] = jnp.broadcast_to(dd[l], (16,))

        def row(i, _):
            di = dsplat[i]
            for r in range(8):
                ebuf[i, pl.ds(16 * r, 16)] = ebuf[i, pl.ds(16 * r, 16)] * di
            return 0
        lax.fori_loop(0, RB, row, 0)
        pltpu.sync_copy(dbufd, dnode_out.at[pl.ds(g, RB)])
        pltpu.sync_copy(ebuf, xs0_out.at[pl.ds(g, RB)])
        return 0
    lax.fori_loop(0, RPT // RB, blk, 0)


_pre = functools.partial(
    pl.kernel,
    out_type=(
        jax.ShapeDtypeStruct((NP,), jnp.float32),
        jax.ShapeDtypeStruct((NP, D), jnp.float32),
    ),
    mesh=_mesh,
    scratch_types=[
        pltpu.MemorySpace.VMEM_SHARED((P,), jnp.float32),
        pltpu.VMEM((NCH, EPC), jnp.int32),
        pltpu.VMEM((EPC,), jnp.float32),
        pltpu.VMEM((RB,), jnp.float32),
        pltpu.VMEM((RB, 16), jnp.float32),
        pltpu.VMEM((RB, D), jnp.float32),
    ],
)(_pre_body)


def _pad_side(a, fill):
    a = a.reshape(NSUB, E // NSUB)
    a = jnp.pad(a, ((0, 0), (0, EPT - E // NSUB)), constant_values=fill)
    return a.reshape(NSUB * NCH, EPC)


def kernel(user_embeddings, item_embeddings, rows, cols):
    dsti = jnp.concatenate([_pad_side(cols + P, 0), _pad_side(rows, 0)])
    srci = jnp.concatenate([_pad_side(rows, P - 1), _pad_side(cols, P - 1)])

    ego = jnp.zeros((NP, D), jnp.float32)
    ego = ego.at[:NU].set(user_embeddings).at[P:P + NI].set(item_embeddings)

    dnode, xs = _pre(ego, srci)
    acc = jnp.zeros((NP, D), jnp.float32)
    for _ in range(NL):
        xs, acc = _layer(xs, ego, dsti, srci, dnode, acc)
    return (acc[:NU], acc[P:P + NI])
